# baseline (speedup 1.0000x reference)
import jax
import jax.numpy as jnp
from jax.experimental import pallas as pl


_EB = 256
_LB = 2048


def _body(x_ref, t_ref, o_ref):
    t = t_ref[...]
    o_ref[...] = x_ref[...] + t.T[None, :, :]


def kernel(x, table):
    b, e, l = x.shape
    grid = (e // _EB, l // _LB)
    return pl.pallas_call(
        _body,
        grid=grid,
        in_specs=[
            pl.BlockSpec((b, _EB, _LB), lambda ei, li: (0, ei, li)),
            pl.BlockSpec((_LB, _EB), lambda ei, li: (li, ei)),
        ],
        out_specs=pl.BlockSpec((b, _EB, _LB), lambda ei, li: (0, ei, li)),
        out_shape=jax.ShapeDtypeStruct(x.shape, x.dtype),
    )(x, table)


# speedup vs baseline: 1.0215x; 1.0215x over previous
import jax
import jax.numpy as jnp
from jax.experimental import pallas as pl


_EB = 128
_LB = 4096


def _body(x_ref, t_ref, o_ref):
    t = t_ref[...]
    o_ref[...] = x_ref[...] + t.T[None, :, :]


def kernel(x, table):
    b, e, l = x.shape
    grid = (e // _EB, l // _LB)
    return pl.pallas_call(
        _body,
        grid=grid,
        in_specs=[
            pl.BlockSpec((b, _EB, _LB), lambda ei, li: (0, ei, li)),
            pl.BlockSpec((_LB, _EB), lambda ei, li: (li, ei)),
        ],
        out_specs=pl.BlockSpec((b, _EB, _LB), lambda ei, li: (0, ei, li)),
        out_shape=jax.ShapeDtypeStruct(x.shape, x.dtype),
    )(x, table)
